# baseline (speedup 1.0000x reference)
import functools

import jax
import jax.numpy as jnp
import numpy as np
from jax import lax
from jax.experimental import pallas as pl
from jax.experimental.pallas import tpu as pltpu
from jax.experimental.pallas import tpu_sc as plsc

L = 16
NC = 1
NS = 16
NW = NC * NS
B = 16384
D = 128
BPW = B // NW
CH = 128
NCHUNK = BPW // CH
NG = CH // L
R_MIN = np.float32(1.0)
R_RANGE = np.float32(4.0)


def _sc_predict(uid3, iid3, user_factor, item_factor, user_bias, item_bias,
                bias16):
    mesh = plsc.VectorSubcoreMesh(core_axis_name="c", subcore_axis_name="s", num_cores=1)

    @functools.partial(
        pl.kernel,
        mesh=mesh,
        compiler_params=pltpu.CompilerParams(needs_layout_passes=False),
        out_type=jax.ShapeDtypeStruct((B,), jnp.float32),
        scratch_types=[
            pltpu.VMEM((NCHUNK, CH), jnp.int32),
            pltpu.VMEM((NCHUNK, CH), jnp.int32),
            pltpu.VMEM((CH, D), jnp.float32),
            pltpu.VMEM((CH, D), jnp.float32),
            pltpu.VMEM((CH, D), jnp.float32),
            pltpu.VMEM((CH, D), jnp.float32),
            pltpu.VMEM((CH, D), jnp.float32),
            pltpu.VMEM((CH, D), jnp.float32),
            pltpu.VMEM((CH,), jnp.float32),
            pltpu.VMEM((CH,), jnp.float32),
            pltpu.VMEM((CH,), jnp.float32),
            pltpu.VMEM((CH,), jnp.float32),
            pltpu.VMEM((CH,), jnp.float32),
            pltpu.VMEM((CH,), jnp.float32),
            pltpu.VMEM((BPW,), jnp.float32),
            pltpu.VMEM((L,), jnp.float32),
            pltpu.SemaphoreType.DMA,
            pltpu.SemaphoreType.DMA,
            pltpu.SemaphoreType.DMA,
            pltpu.SemaphoreType.DMA,
            pltpu.SemaphoreType.DMA,
            pltpu.SemaphoreType.DMA,
            pltpu.SemaphoreType.DMA,
            pltpu.SemaphoreType.DMA,
            pltpu.SemaphoreType.DMA,
            pltpu.SemaphoreType.DMA,
            pltpu.SemaphoreType.DMA,
            pltpu.SemaphoreType.DMA,
        ],
    )
    def k(uid_h, iid_h, uf_h, itf_h, ub_h, ib_h, bias_h, out_h,
          uid_v, iid_v, ufr0, itr0, ufr1, itr1, ufr2, itr2,
          ubr0, ibr0, ubr1, ibr1, ubr2, ibr2,
          outv, biasv, s0, s1, s2, s3, s4, s5, s6, s7, s8, s9, s10, s11):
        wid = lax.axis_index("s") * NC + lax.axis_index("c")
        pltpu.sync_copy(uid_h.at[wid], uid_v)
        pltpu.sync_copy(iid_h.at[wid], iid_v)
        pltpu.sync_copy(bias_h, biasv)
        bvec = biasv[...]
        lanes = lax.iota(jnp.int32, L)
        eqs = [lanes == r for r in range(L)]
        bufs = [
            (ufr0, itr0, ubr0, ibr0, (s0, s1, s2, s3)),
            (ufr1, itr1, ubr1, ibr1, (s4, s5, s6, s7)),
            (ufr2, itr2, ubr2, ibr2, (s8, s9, s10, s11)),
        ]
        NBUF = len(bufs)

        def issue(c, buf):
            ufr, itr, ubr, ibr, ss = buf
            return [
                pltpu.async_copy(uf_h.at[uid_v.at[c]], ufr, ss[0]),
                pltpu.async_copy(itf_h.at[iid_v.at[c]], itr, ss[1]),
                pltpu.async_copy(ub_h.at[uid_v.at[c]], ubr, ss[2]),
                pltpu.async_copy(ib_h.at[iid_v.at[c]], ibr, ss[3]),
            ]

        pending = {c: issue(c, bufs[c]) for c in range(min(NBUF, NCHUNK))}
        for c in range(NCHUNK):
            ufr, itr, ubr, ibr, _ = bufs[c % NBUF]
            for cp in pending.pop(c):
                cp.wait()

            def group_body(g, carry2, ufr=ufr, itr=itr, ubr=ubr, ibr=ibr,
                           c=c):
                rowbase = g * L

                @plsc.parallel_loop(0, L, carry=jnp.zeros((L,), jnp.float32),
                                    unroll=4)
                def rowsum(r, cur):
                    row = rowbase + r
                    acc = ufr[row, pl.ds(0, L)] * itr[row, pl.ds(0, L)]
                    for kk in range(1, D // L):
                        acc = acc + (ufr[row, pl.ds(kk * L, L)] *
                                     itr[row, pl.ds(kk * L, L)])
                    return jnp.where(lanes == r, jnp.sum(acc), cur)

                off = pl.multiple_of(rowbase, L)
                logits = (rowsum + ubr[pl.ds(off, L)] + ibr[pl.ds(off, L)]
                          + bvec)
                e = jnp.exp(-logits)
                pred = R_MIN + R_RANGE / (np.float32(1.0) + e)
                oout = pl.multiple_of(c * CH + rowbase, L)
                outv[pl.ds(oout, L)] = pred
                return carry2

            lax.fori_loop(0, NG, group_body, 0)
            if c + NBUF < NCHUNK:
                pending[c + NBUF] = issue(c + NBUF, bufs[(c + NBUF) % NBUF])

        obase = pl.multiple_of(wid * BPW, BPW)
        pltpu.sync_copy(outv, out_h.at[pl.ds(obase, BPW)])

    return k(uid3, iid3, user_factor, item_factor, user_bias, item_bias,
             bias16)


def kernel(user_id, item_id, user_factor, item_factor, user_bias, item_bias,
           bias):
    uid3 = user_id.reshape(NW, NCHUNK, CH)
    iid3 = item_id.reshape(NW, NCHUNK, CH)
    bias16 = jnp.broadcast_to(jnp.reshape(bias, (1,)), (L,))
    return _sc_predict(uid3, iid3, user_factor, item_factor, user_bias,
                       item_bias, bias16)


# speedup vs baseline: 1.1518x; 1.1518x over previous
import functools

import jax
import jax.numpy as jnp
import numpy as np
from jax import lax
from jax.experimental import pallas as pl
from jax.experimental.pallas import tpu as pltpu
from jax.experimental.pallas import tpu_sc as plsc

L = 16
NC = 2
NS = 16
NW = NC * NS
B = 16384
D = 128
BPW = B // NW
CH = 128
NCHUNK = BPW // CH
NG = CH // L
R_MIN = np.float32(1.0)
R_RANGE = np.float32(4.0)


def _sc_predict(uid3, iid3, user_factor, item_factor, user_bias, item_bias,
                bias16):
    mesh = plsc.VectorSubcoreMesh(core_axis_name="c", subcore_axis_name="s")

    @functools.partial(
        pl.kernel,
        mesh=mesh,
        compiler_params=pltpu.CompilerParams(needs_layout_passes=False),
        out_type=jax.ShapeDtypeStruct((B,), jnp.float32),
        scratch_types=[
            pltpu.VMEM((NCHUNK, CH), jnp.int32),
            pltpu.VMEM((NCHUNK, CH), jnp.int32),
            pltpu.VMEM((CH, D), jnp.float32),
            pltpu.VMEM((CH, D), jnp.float32),
            pltpu.VMEM((CH, D), jnp.float32),
            pltpu.VMEM((CH, D), jnp.float32),
            pltpu.VMEM((CH, D), jnp.float32),
            pltpu.VMEM((CH, D), jnp.float32),
            pltpu.VMEM((CH,), jnp.float32),
            pltpu.VMEM((CH,), jnp.float32),
            pltpu.VMEM((CH,), jnp.float32),
            pltpu.VMEM((CH,), jnp.float32),
            pltpu.VMEM((CH,), jnp.float32),
            pltpu.VMEM((CH,), jnp.float32),
            pltpu.VMEM((BPW,), jnp.float32),
            pltpu.VMEM((L,), jnp.float32),
            pltpu.SemaphoreType.DMA,
            pltpu.SemaphoreType.DMA,
            pltpu.SemaphoreType.DMA,
            pltpu.SemaphoreType.DMA,
            pltpu.SemaphoreType.DMA,
            pltpu.SemaphoreType.DMA,
            pltpu.SemaphoreType.DMA,
            pltpu.SemaphoreType.DMA,
            pltpu.SemaphoreType.DMA,
            pltpu.SemaphoreType.DMA,
            pltpu.SemaphoreType.DMA,
            pltpu.SemaphoreType.DMA,
        ],
    )
    def k(uid_h, iid_h, uf_h, itf_h, ub_h, ib_h, bias_h, out_h,
          uid_v, iid_v, ufr0, itr0, ufr1, itr1, ufr2, itr2,
          ubr0, ibr0, ubr1, ibr1, ubr2, ibr2,
          outv, biasv, s0, s1, s2, s3, s4, s5, s6, s7, s8, s9, s10, s11):
        wid = lax.axis_index("s") * NC + lax.axis_index("c")
        pltpu.sync_copy(uid_h.at[wid], uid_v)
        pltpu.sync_copy(iid_h.at[wid], iid_v)
        pltpu.sync_copy(bias_h, biasv)
        bvec = biasv[...]
        lanes = lax.iota(jnp.int32, L)
        eqs = [lanes == r for r in range(L)]
        bufs = [
            (ufr0, itr0, ubr0, ibr0, (s0, s1, s2, s3)),
            (ufr1, itr1, ubr1, ibr1, (s4, s5, s6, s7)),
            (ufr2, itr2, ubr2, ibr2, (s8, s9, s10, s11)),
        ]
        NBUF = len(bufs)

        def issue(c, buf):
            ufr, itr, ubr, ibr, ss = buf
            return [
                pltpu.async_copy(uf_h.at[uid_v.at[c]], ufr, ss[0]),
                pltpu.async_copy(itf_h.at[iid_v.at[c]], itr, ss[1]),
                pltpu.async_copy(ub_h.at[uid_v.at[c]], ubr, ss[2]),
                pltpu.async_copy(ib_h.at[iid_v.at[c]], ibr, ss[3]),
            ]

        pending = {c: issue(c, bufs[c]) for c in range(min(NBUF, NCHUNK))}
        for c in range(NCHUNK):
            ufr, itr, ubr, ibr, _ = bufs[c % NBUF]
            for cp in pending.pop(c):
                cp.wait()

            def group_body(g, carry2, ufr=ufr, itr=itr, ubr=ubr, ibr=ibr,
                           c=c):
                rowbase = g * L

                @plsc.parallel_loop(0, L, carry=jnp.zeros((L,), jnp.float32),
                                    unroll=4)
                def rowsum(r, cur):
                    row = rowbase + r
                    acc = ufr[row, pl.ds(0, L)] * itr[row, pl.ds(0, L)]
                    for kk in range(1, D // L):
                        acc = acc + (ufr[row, pl.ds(kk * L, L)] *
                                     itr[row, pl.ds(kk * L, L)])
                    return jnp.where(lanes == r, jnp.sum(acc), cur)

                off = pl.multiple_of(rowbase, L)
                logits = (rowsum + ubr[pl.ds(off, L)] + ibr[pl.ds(off, L)]
                          + bvec)
                e = jnp.exp(-logits)
                pred = R_MIN + R_RANGE / (np.float32(1.0) + e)
                oout = pl.multiple_of(c * CH + rowbase, L)
                outv[pl.ds(oout, L)] = pred
                return carry2

            lax.fori_loop(0, NG, group_body, 0)
            if c + NBUF < NCHUNK:
                pending[c + NBUF] = issue(c + NBUF, bufs[(c + NBUF) % NBUF])

        obase = pl.multiple_of(wid * BPW, BPW)
        pltpu.sync_copy(outv, out_h.at[pl.ds(obase, BPW)])

    return k(uid3, iid3, user_factor, item_factor, user_bias, item_bias,
             bias16)


def kernel(user_id, item_id, user_factor, item_factor, user_bias, item_bias,
           bias):
    uid3 = user_id.reshape(NW, NCHUNK, CH)
    iid3 = item_id.reshape(NW, NCHUNK, CH)
    bias16 = jnp.broadcast_to(jnp.reshape(bias, (1,)), (L,))
    return _sc_predict(uid3, iid3, user_factor, item_factor, user_bias,
                       item_bias, bias16)
